# Initial kernel scaffold; baseline (speedup 1.0000x reference)
import jax
import jax.numpy as jnp
from jax.experimental import pallas as pl


def kernel(K, alpha_raw, bin_edges):
    raise NotImplementedError("write your pallas kernel here")


# speedup vs baseline: 3.6408x; 3.6408x over previous
import functools

import jax
import jax.numpy as jnp
from jax import lax
from jax.experimental import pallas as pl
from jax.experimental.pallas import tpu as pltpu
from jax.experimental.pallas import tpu_sc as plsc

_LANES = 16
_NWORKERS = 32
_CHUNK = 16384


def _sc_bin_table(N, *, num_bins, num_edges):
    per_tile = N // _NWORKERS
    n_chunks = per_tile // _CHUNK
    n_inner = num_edges - 2

    mesh = plsc.VectorSubcoreMesh(core_axis_name="c", subcore_axis_name="s")

    @functools.partial(
        pl.kernel,
        out_type=jax.ShapeDtypeStruct((N,), jnp.float32),
        mesh=mesh,
        scratch_types=[
            pltpu.VMEM((_LANES,), jnp.float32),
            pltpu.VMEM((_LANES,), jnp.float32),
            pltpu.VMEM((_LANES,), jnp.float32),
            pltpu.VMEM((2, _CHUNK), jnp.float32),
            pltpu.VMEM((2, _CHUNK), jnp.float32),
            pltpu.SemaphoreType.DMA,
            pltpu.SemaphoreType.DMA,
            pltpu.SemaphoreType.DMA,
            pltpu.SemaphoreType.DMA,
        ],
    )
    def sc_run(k_hbm, alpha_hbm, edges_hbm, out_hbm,
               alpha_v, tbl_v, edges_v, ibuf, obuf,
               isem0, isem1, osem0, osem1):
        cid = lax.axis_index("c")
        sid = lax.axis_index("s")
        wid = sid * 2 + cid
        base = wid * per_tile

        pltpu.sync_copy(alpha_hbm, alpha_v)
        pltpu.sync_copy(edges_hbm, edges_v)
        a = alpha_v[...]
        tbl = 1.0 / (1.0 + jnp.exp(-a))
        tbl_v[...] = tbl
        ev = edges_v[...]

        s0 = tbl[0]
        deltas = [tbl[j + 1] - tbl[j] for j in range(num_bins - 1)]
        inner = [ev[j + 1] for j in range(n_inner)]

        isems = [isem0, isem1]
        osems = [osem0, osem1]
        zero = jnp.zeros((_LANES,), jnp.float32)

        def start_in(g):
            return pltpu.async_copy(
                k_hbm.at[pl.ds(base + g * _CHUNK, _CHUNK)],
                ibuf.at[g % 2], isems[g % 2])

        def start_out(g):
            return pltpu.async_copy(
                obuf.at[g % 2],
                out_hbm.at[pl.ds(base + g * _CHUNK, _CHUNK)],
                osems[g % 2])

        in_flight = [None, None]
        out_flight = [None, None]
        in_flight[0] = start_in(0)
        for g in range(n_chunks):
            p = g % 2
            if g + 1 < n_chunks:
                in_flight[(g + 1) % 2] = start_in(g + 1)
            in_flight[p].wait()
            if out_flight[p] is not None:
                out_flight[p].wait()
            ib = ibuf.at[p]
            ob = obuf.at[p]

            @plsc.parallel_loop(0, _CHUNK, _LANES, unroll=4)
            def _(i):
                x = ib[pl.ds(i, _LANES)]
                acc = jnp.full((_LANES,), 0.0, jnp.float32) + s0
                for j in range(n_inner):
                    acc = acc + jnp.where(x > inner[j], deltas[j], zero)
                ob[pl.ds(i, _LANES)] = acc

            out_flight[p] = start_out(g)
        for h in out_flight:
            if h is not None:
                h.wait()

    return sc_run


def kernel(K, alpha_raw, bin_edges):
    num_bins = alpha_raw.shape[0]
    num_edges = bin_edges.shape[0]
    alpha_pad = jnp.zeros((_LANES,), jnp.float32).at[:num_bins].set(alpha_raw)
    edges_pad = jnp.zeros((_LANES,), jnp.float32).at[:num_edges].set(bin_edges)
    run = _sc_bin_table(K.shape[0],
                        num_bins=num_bins, num_edges=num_edges)
    return run(K, alpha_pad, edges_pad)
